# baseline (speedup 1.0000x reference)
import functools

import jax
import jax.numpy as jnp
from jax import lax
from jax.experimental import pallas as pl
from jax.experimental.pallas import tpu as pltpu
from jax.experimental.pallas import tpu_sc as plsc

B = 128
S = 512
D = 768
L = 16
NC = 2
NS = 16
NW = NC * NS
P_PER_W = S // NW
DV = D // L

_MAGIC = 0x5F3759DF


def _rsqrt_newton(x):
    i = plsc.bitcast(x, jnp.int32)
    y = plsc.bitcast(_MAGIC - (i >> 1), jnp.float32)
    for _ in range(3):
        y = y * (1.5 - 0.5 * x * y * y)
    return y


def _body(word_hbm, x_hbm, pos_hbm, gamma_hbm, beta_hbm, out_hbm,
          pos_v, g_v, bb_v, idx_v, rows_v, gsem):
    wid = lax.axis_index("s") * NC + lax.axis_index("c")
    p0 = wid * P_PER_W

    pltpu.sync_copy(pos_hbm.at[pl.ds(p0, P_PER_W)], pos_v)
    pltpu.sync_copy(gamma_hbm, g_v)
    pltpu.sync_copy(beta_hbm, bb_v)

    @pl.loop(0, B)
    def _batch(b):
        base = b * S + p0
        pltpu.sync_copy(x_hbm.at[pl.ds(base, P_PER_W)], idx_v)
        pltpu.async_copy(word_hbm.at[idx_v], rows_v, gsem).wait()

        @pl.loop(0, P_PER_W)
        def _row(r):
            sum_v = jnp.zeros((L,), jnp.float32)
            sq_v = jnp.zeros((L,), jnp.float32)
            for j in range(DV):
                sl = pl.ds(j * L, L)
                e = rows_v[r, sl] + pos_v[r, sl]
                rows_v[r, sl] = e
                sum_v = sum_v + e
                sq_v = sq_v + e * e
            mean = jnp.sum(sum_v) * (1.0 / D)
            var = jnp.sum(sq_v) * (1.0 / D) - mean * mean
            inv_v = _rsqrt_newton(jnp.full((L,), var + 1e-12, jnp.float32))
            mean_v = jnp.full((L,), mean, jnp.float32)
            for j in range(DV):
                sl = pl.ds(j * L, L)
                e = rows_v[r, sl]
                rows_v[r, sl] = (e - mean_v) * inv_v * g_v[sl] + bb_v[sl]

        pltpu.sync_copy(rows_v, out_hbm.at[pl.ds(base, P_PER_W)])


@functools.partial(
    pl.kernel,
    out_type=jax.ShapeDtypeStruct((B * S, D), jnp.float32),
    mesh=plsc.VectorSubcoreMesh(core_axis_name="c", subcore_axis_name="s",
                                num_cores=NC, num_subcores=NS),
    compiler_params=pltpu.CompilerParams(needs_layout_passes=False),
    scratch_types=[
        pltpu.VMEM((P_PER_W, D), jnp.float32),
        pltpu.VMEM((D,), jnp.float32),
        pltpu.VMEM((D,), jnp.float32),
        pltpu.VMEM((P_PER_W,), jnp.int32),
        pltpu.VMEM((P_PER_W, D), jnp.float32),
        pltpu.SemaphoreType.DMA,
    ],
)
def _emb_ln(word_hbm, x_hbm, pos_hbm, gamma_hbm, beta_hbm, out_hbm,
            pos_v, g_v, bb_v, idx_v, rows_v, gsem):
    _body(word_hbm, x_hbm, pos_hbm, gamma_hbm, beta_hbm, out_hbm,
          pos_v, g_v, bb_v, idx_v, rows_v, gsem)


def kernel(x, word_table, pos_table, gamma, beta):
    xf = x.reshape(-1).astype(jnp.int32)
    out = _emb_ln(word_table, xf, pos_table, gamma, beta)
    return out.reshape(B, S, D)
